# baseline (speedup 1.0000x reference)
import functools
import jax
import jax.numpy as jnp
from jax import lax
from jax.experimental import pallas as pl
from jax.experimental.pallas import tpu as pltpu
from jax.experimental.pallas import tpu_sc as plsc

NN = 10000
NP = 10240
DD = 128
CHUNK = 96
NCORE = 2
NSUB = 16
NW = NCORE * NSUB
ROWS_PER_TILE = NP // NSUB
ZSRC = 80
WB = 128


def _spmm_body(per_tile, h_hbm, pk_hbm, w_hbm, out_hbm,
               pk_v, w_v, rows0, rows1, col0, col1, rix0, rix1,
               rtail_v, acc_sh, g0, g1, s0, s1):
    c = lax.axis_index("c")
    s = lax.axis_index("s")
    nfull = per_tile // CHUNK
    tail = per_tile % CHUNK
    assert nfull % 2 == 0 and tail in (0, 16)
    wid = s * NCORE + c

    def unpack(base, n, col_dst, row_dst):
        for g in range(n // 16):
            pk = pk_v[pl.ds(base + g * 16, 16)]
            col_dst[pl.ds(g * 16, 16)] = pk >> 16
            row_dst[pl.ds(g * 16, 16)] = pk & 0xFFFF

    def scale(rows, wbase, n):
        def grp(g, _):
            w16 = w_v[pl.ds(wbase + g * 16, 16)]
            for l in range(16):
                w = w16[l]
                e = g * 16 + l
                for k in range(DD // 16):
                    rows[e, pl.ds(k * 16, 16)] = rows[e, pl.ds(k * 16, 16)] * w
            return 0
        lax.fori_loop(0, n // 16, grp, 0, unroll=2)

    def zbody(i, _):
        for g in range(DD // 16):
            rows0[i, pl.ds(g * 16, 16)] = jnp.zeros((16,), jnp.float32)
        return 0
    lax.fori_loop(0, ZSRC, zbody, 0)
    for t in range(ROWS_PER_TILE // ZSRC):
        pltpu.async_copy(
            rows0.at[pl.ds(0, ZSRC)],
            acc_sh.at[pl.ds(s * ROWS_PER_TILE + t * ZSRC, ZSRC)], g0)
    pltpu.async_copy(pk_hbm.at[wid], pk_v, g1)
    pltpu.async_copy(w_hbm.at[wid], w_v, g1)
    for t in range(ROWS_PER_TILE // ZSRC):
        pltpu.make_async_copy(
            rows0.at[pl.ds(0, ZSRC)],
            acc_sh.at[pl.ds(s * ROWS_PER_TILE + t * ZSRC, ZSRC)], g0).wait()
    pltpu.make_async_copy(pk_hbm.at[wid], pk_v, g1).wait()
    pltpu.make_async_copy(w_hbm.at[wid], w_v, g1).wait()
    plsc.subcore_barrier()

    unpack(0, CHUNK, col0, rix0)
    pltpu.async_copy(h_hbm.at[col0], rows0, g0)

    def pair(i, _):
        jA = 2 * i
        pltpu.make_async_copy(h_hbm.at[col0], rows0, g0).wait()

        @pl.when(i > 0)
        def _():
            pltpu.make_async_copy(rows1, acc_sh.at[rix1], s1).wait()
        unpack((jA + 1) * CHUNK, CHUNK, col1, rix1)
        pltpu.async_copy(h_hbm.at[col1], rows1, g1)
        scale(rows0, jA * CHUNK, CHUNK)
        pltpu.async_copy(rows0, acc_sh.at[rix0], s0, add=True)

        jB = jA + 1
        pltpu.make_async_copy(h_hbm.at[col1], rows1, g1).wait()

        @pl.when(i < nfull // 2 - 1)
        def _():
            pltpu.make_async_copy(rows0, acc_sh.at[rix0], s0).wait()
            unpack((jB + 1) * CHUNK, CHUNK, col0, rix0)
            pltpu.async_copy(h_hbm.at[col0], rows0, g0)
        scale(rows1, jB * CHUNK, CHUNK)
        pltpu.async_copy(rows1, acc_sh.at[rix1], s1, add=True)
        return 0
    lax.fori_loop(0, nfull // 2, pair, 0)

    pltpu.make_async_copy(rows0, acc_sh.at[rix0], s0).wait()
    if tail:
        base = nfull * CHUNK
        pk = pk_v[pl.ds(base, 16)]
        col0[pl.ds(0, 16)] = pk >> 16
        rtail_v[pl.ds(0, 16)] = pk & 0xFFFF
        pltpu.async_copy(h_hbm.at[col0.at[pl.ds(0, tail)]],
                         rows0.at[pl.ds(0, tail)], g0).wait()
        scale(rows0, base, tail)
        pltpu.async_copy(rows0.at[pl.ds(0, tail)],
                         acc_sh.at[rtail_v], s0, add=True).wait()
    pltpu.make_async_copy(rows1, acc_sh.at[rix1], s1).wait()

    plsc.subcore_barrier()
    for t in range(ROWS_PER_TILE // WB):
        off = s * ROWS_PER_TILE + t * WB
        pltpu.async_copy(acc_sh.at[pl.ds(off, WB)],
                         out_hbm.at[c, pl.ds(off, WB)], g0)
    for t in range(ROWS_PER_TILE // WB):
        off = s * ROWS_PER_TILE + t * WB
        pltpu.make_async_copy(acc_sh.at[pl.ds(off, WB)],
                              out_hbm.at[c, pl.ds(off, WB)], g0).wait()


def _make_spmm(per_tile):
    tail = per_tile % CHUNK
    mesh = plsc.VectorSubcoreMesh(core_axis_name="c", subcore_axis_name="s")
    return functools.partial(
        pl.kernel,
        out_type=jax.ShapeDtypeStruct((NCORE, NP, DD), jnp.float32),
        mesh=mesh,
        scratch_types=[
            pltpu.VMEM((per_tile,), jnp.int32),
            pltpu.VMEM((per_tile,), jnp.float32),
            pltpu.VMEM((CHUNK, DD), jnp.float32),
            pltpu.VMEM((CHUNK, DD), jnp.float32),
            pltpu.VMEM((CHUNK,), jnp.int32),
            pltpu.VMEM((CHUNK,), jnp.int32),
            pltpu.VMEM((CHUNK,), jnp.int32),
            pltpu.VMEM((CHUNK,), jnp.int32),
            pltpu.VMEM((max(tail, 16),), jnp.int32),
            pltpu.VMEM_SHARED((NP, DD), jnp.float32),
            pltpu.SemaphoreType.DMA,
            pltpu.SemaphoreType.DMA,
            pltpu.SemaphoreType.DMA,
            pltpu.SemaphoreType.DMA,
        ],
    )(functools.partial(_spmm_body, per_tile))


_BLK = 1000


def _mlp1_body(x_ref, w1_ref, b1_ref, wf_ref, bf_ref, o_ref):
    h = jnp.maximum(
        jnp.dot(x_ref[...], w1_ref[...], preferred_element_type=jnp.float32)
        + b1_ref[...], 0.0)
    o_ref[...] = (
        jnp.dot(h, wf_ref[...], preferred_element_type=jnp.float32)
        + bf_ref[...])


def _mlp2_body(p_ref, w2_ref, b2_ref, o_ref):
    h = jnp.maximum(p_ref[0] + p_ref[1], 0.0)
    o_ref[...] = (
        jnp.dot(h, w2_ref[...], preferred_element_type=jnp.float32)
        + b2_ref[...])


def _add_body(q_ref, o_ref):
    o_ref[...] = q_ref[0] + q_ref[1]


def _mlp1(x, W1, b1, Wf, bf):
    return pl.pallas_call(
        _mlp1_body,
        grid=(NN // _BLK,),
        in_specs=[
            pl.BlockSpec((_BLK, DD), lambda i: (i, 0)),
            pl.BlockSpec((DD, DD), lambda i: (0, 0)),
            pl.BlockSpec((1, DD), lambda i: (0, 0)),
            pl.BlockSpec((DD, DD), lambda i: (0, 0)),
            pl.BlockSpec((1, DD), lambda i: (0, 0)),
        ],
        out_specs=pl.BlockSpec((_BLK, DD), lambda i: (i, 0)),
        out_shape=jax.ShapeDtypeStruct((NN, DD), jnp.float32),
    )(x, W1, b1.reshape(1, DD), Wf, bf.reshape(1, DD))


def _mlp2(p, W2, b2):
    return pl.pallas_call(
        _mlp2_body,
        grid=(NN // _BLK,),
        in_specs=[
            pl.BlockSpec((NCORE, _BLK, DD), lambda i: (0, i, 0)),
            pl.BlockSpec((DD, DD), lambda i: (0, 0)),
            pl.BlockSpec((1, DD), lambda i: (0, 0)),
        ],
        out_specs=pl.BlockSpec((_BLK, DD), lambda i: (i, 0)),
        out_shape=jax.ShapeDtypeStruct((NN, DD), jnp.float32),
    )(p, W2, b2.reshape(1, DD))


def _padd(q):
    return pl.pallas_call(
        _add_body,
        grid=(NN // _BLK,),
        in_specs=[pl.BlockSpec((NCORE, _BLK, DD), lambda i: (0, i, 0))],
        out_specs=pl.BlockSpec((_BLK, DD), lambda i: (i, 0)),
        out_shape=jax.ShapeDtypeStruct((NN, DD), jnp.float32),
    )(q)


def kernel(x_node, edge_weight, W1, b1, Wf, bf, W2, b2, edge_index):
    E = edge_weight.shape[0]
    assert E % NW == 0
    per_tile = E // NW

    pk_flat = (edge_index[1] << 16) | edge_index[0]
    pk2 = pk_flat.reshape(NW, per_tile)
    w2 = edge_weight.reshape(NW, per_tile)

    spmm = _make_spmm(per_tile)

    h1 = _mlp1(x_node, W1, b1, Wf, bf)
    p = spmm(h1, pk2, w2)
    h2 = _mlp2(p, W2, b2)
    q = spmm(h2, pk2, w2)
    return _padd(q)


# speedup vs baseline: 1.0031x; 1.0031x over previous
import functools
import jax
import jax.numpy as jnp
from jax import lax
from jax.experimental import pallas as pl
from jax.experimental.pallas import tpu as pltpu
from jax.experimental.pallas import tpu_sc as plsc

NN = 10000
NP = 10240
DD = 128
CHUNK = 96
NCORE = 2
NSUB = 16
NW = NCORE * NSUB
ROWS_PER_TILE = NP // NSUB
ZSRC = 80
WB = 128


def _spmm_body(per_tile, h_hbm, pk_hbm, w_hbm, out_hbm,
               pk_v, w_v, rows0, rows1, col0, col1, rix0, rix1,
               rtail_v, acc_sh, g0, g1, s0, s1):
    c = lax.axis_index("c")
    s = lax.axis_index("s")
    nfull = per_tile // CHUNK
    tail = per_tile % CHUNK
    assert nfull % 2 == 0 and tail in (0, 16)
    wid = s * NCORE + c

    def unpack(base, n, col_dst, row_dst):
        for g in range(n // 16):
            pk = pk_v[pl.ds(base + g * 16, 16)]
            col_dst[pl.ds(g * 16, 16)] = pk >> 16
            row_dst[pl.ds(g * 16, 16)] = pk & 0xFFFF

    def scale(rows, wbase, n):
        def grp(g, _):
            w16 = w_v[pl.ds(wbase + g * 16, 16)]
            for l in range(16):
                w = w16[l]
                e = g * 16 + l
                for k in range(DD // 16):
                    rows[e, pl.ds(k * 16, 16)] = rows[e, pl.ds(k * 16, 16)] * w
            return 0
        lax.fori_loop(0, n // 16, grp, 0)

    def zbody(i, _):
        for g in range(DD // 16):
            rows0[i, pl.ds(g * 16, 16)] = jnp.zeros((16,), jnp.float32)
        return 0
    lax.fori_loop(0, ZSRC, zbody, 0)
    for t in range(ROWS_PER_TILE // ZSRC):
        pltpu.async_copy(
            rows0.at[pl.ds(0, ZSRC)],
            acc_sh.at[pl.ds(s * ROWS_PER_TILE + t * ZSRC, ZSRC)], g0)
    pltpu.async_copy(pk_hbm.at[wid], pk_v, g1)
    pltpu.async_copy(w_hbm.at[wid], w_v, g1)
    for t in range(ROWS_PER_TILE // ZSRC):
        pltpu.make_async_copy(
            rows0.at[pl.ds(0, ZSRC)],
            acc_sh.at[pl.ds(s * ROWS_PER_TILE + t * ZSRC, ZSRC)], g0).wait()
    pltpu.make_async_copy(pk_hbm.at[wid], pk_v, g1).wait()
    pltpu.make_async_copy(w_hbm.at[wid], w_v, g1).wait()
    plsc.subcore_barrier()

    unpack(0, CHUNK, col0, rix0)
    pltpu.async_copy(h_hbm.at[col0], rows0, g0)

    def pair(i, _):
        jA = 2 * i
        pltpu.make_async_copy(h_hbm.at[col0], rows0, g0).wait()

        @pl.when(i > 0)
        def _():
            pltpu.make_async_copy(rows1, acc_sh.at[rix1], s1).wait()
        unpack((jA + 1) * CHUNK, CHUNK, col1, rix1)
        pltpu.async_copy(h_hbm.at[col1], rows1, g1)
        scale(rows0, jA * CHUNK, CHUNK)
        pltpu.async_copy(rows0, acc_sh.at[rix0], s0, add=True)

        jB = jA + 1
        pltpu.make_async_copy(h_hbm.at[col1], rows1, g1).wait()

        @pl.when(i < nfull // 2 - 1)
        def _():
            pltpu.make_async_copy(rows0, acc_sh.at[rix0], s0).wait()
            unpack((jB + 1) * CHUNK, CHUNK, col0, rix0)
            pltpu.async_copy(h_hbm.at[col0], rows0, g0)
        scale(rows1, jB * CHUNK, CHUNK)
        pltpu.async_copy(rows1, acc_sh.at[rix1], s1, add=True)
        return 0
    lax.fori_loop(0, nfull // 2, pair, 0)

    pltpu.make_async_copy(rows0, acc_sh.at[rix0], s0).wait()
    if tail:
        base = nfull * CHUNK
        pk = pk_v[pl.ds(base, 16)]
        col0[pl.ds(0, 16)] = pk >> 16
        rtail_v[pl.ds(0, 16)] = pk & 0xFFFF
        pltpu.async_copy(h_hbm.at[col0.at[pl.ds(0, tail)]],
                         rows0.at[pl.ds(0, tail)], g0).wait()
        scale(rows0, base, tail)
        pltpu.async_copy(rows0.at[pl.ds(0, tail)],
                         acc_sh.at[rtail_v], s0, add=True).wait()
    pltpu.make_async_copy(rows1, acc_sh.at[rix1], s1).wait()

    plsc.subcore_barrier()
    for t in range(ROWS_PER_TILE // WB):
        off = s * ROWS_PER_TILE + t * WB
        pltpu.async_copy(acc_sh.at[pl.ds(off, WB)],
                         out_hbm.at[c, pl.ds(off, WB)], g0)
    for t in range(ROWS_PER_TILE // WB):
        off = s * ROWS_PER_TILE + t * WB
        pltpu.make_async_copy(acc_sh.at[pl.ds(off, WB)],
                              out_hbm.at[c, pl.ds(off, WB)], g0).wait()


def _make_spmm(per_tile):
    tail = per_tile % CHUNK
    mesh = plsc.VectorSubcoreMesh(core_axis_name="c", subcore_axis_name="s")
    return functools.partial(
        pl.kernel,
        out_type=jax.ShapeDtypeStruct((NCORE, NP, DD), jnp.float32),
        mesh=mesh,
        scratch_types=[
            pltpu.VMEM((per_tile,), jnp.int32),
            pltpu.VMEM((per_tile,), jnp.float32),
            pltpu.VMEM((CHUNK, DD), jnp.float32),
            pltpu.VMEM((CHUNK, DD), jnp.float32),
            pltpu.VMEM((CHUNK,), jnp.int32),
            pltpu.VMEM((CHUNK,), jnp.int32),
            pltpu.VMEM((CHUNK,), jnp.int32),
            pltpu.VMEM((CHUNK,), jnp.int32),
            pltpu.VMEM((max(tail, 16),), jnp.int32),
            pltpu.VMEM_SHARED((NP, DD), jnp.float32),
            pltpu.SemaphoreType.DMA,
            pltpu.SemaphoreType.DMA,
            pltpu.SemaphoreType.DMA,
            pltpu.SemaphoreType.DMA,
        ],
    )(functools.partial(_spmm_body, per_tile))


_BLK = 1000


def _mlp1_body(x_ref, w1_ref, b1_ref, wf_ref, bf_ref, o_ref):
    h = jnp.maximum(
        jnp.dot(x_ref[...], w1_ref[...], preferred_element_type=jnp.float32)
        + b1_ref[...], 0.0)
    o_ref[...] = (
        jnp.dot(h, wf_ref[...], preferred_element_type=jnp.float32)
        + bf_ref[...])


def _mlp2_body(p_ref, w2_ref, b2_ref, o_ref):
    h = jnp.maximum(p_ref[0] + p_ref[1], 0.0)
    o_ref[...] = (
        jnp.dot(h, w2_ref[...], preferred_element_type=jnp.float32)
        + b2_ref[...])


def _add_body(q_ref, o_ref):
    o_ref[...] = q_ref[0] + q_ref[1]


def _mlp1(x, W1, b1, Wf, bf):
    return pl.pallas_call(
        _mlp1_body,
        grid=(NN // _BLK,),
        in_specs=[
            pl.BlockSpec((_BLK, DD), lambda i: (i, 0)),
            pl.BlockSpec((DD, DD), lambda i: (0, 0)),
            pl.BlockSpec((1, DD), lambda i: (0, 0)),
            pl.BlockSpec((DD, DD), lambda i: (0, 0)),
            pl.BlockSpec((1, DD), lambda i: (0, 0)),
        ],
        out_specs=pl.BlockSpec((_BLK, DD), lambda i: (i, 0)),
        out_shape=jax.ShapeDtypeStruct((NN, DD), jnp.float32),
    )(x, W1, b1.reshape(1, DD), Wf, bf.reshape(1, DD))


def _mlp2(p, W2, b2):
    return pl.pallas_call(
        _mlp2_body,
        grid=(NN // _BLK,),
        in_specs=[
            pl.BlockSpec((NCORE, _BLK, DD), lambda i: (0, i, 0)),
            pl.BlockSpec((DD, DD), lambda i: (0, 0)),
            pl.BlockSpec((1, DD), lambda i: (0, 0)),
        ],
        out_specs=pl.BlockSpec((_BLK, DD), lambda i: (i, 0)),
        out_shape=jax.ShapeDtypeStruct((NN, DD), jnp.float32),
    )(p, W2, b2.reshape(1, DD))


def _padd(q):
    return pl.pallas_call(
        _add_body,
        grid=(NN // _BLK,),
        in_specs=[pl.BlockSpec((NCORE, _BLK, DD), lambda i: (0, i, 0))],
        out_specs=pl.BlockSpec((_BLK, DD), lambda i: (i, 0)),
        out_shape=jax.ShapeDtypeStruct((NN, DD), jnp.float32),
    )(q)


def kernel(x_node, edge_weight, W1, b1, Wf, bf, W2, b2, edge_index):
    E = edge_weight.shape[0]
    assert E % NW == 0
    per_tile = E // NW

    pk_flat = (edge_index[1] << 16) | edge_index[0]
    pk2 = pk_flat.reshape(NW, per_tile)
    w2 = edge_weight.reshape(NW, per_tile)

    spmm = _make_spmm(per_tile)

    h1 = _mlp1(x_node, W1, b1, Wf, bf)
    p = spmm(h1, pk2, w2)
    h2 = _mlp2(p, W2, b2)
    q = spmm(h2, pk2, w2)
    return _padd(q)
